# baseline (speedup 1.0000x reference)
import jax
import jax.numpy as jnp
from jax.experimental import pallas as pl

_BM = 256


def _body(l_ref, featb_ref, feat_blk_ref, wc_ref, bias_ref, o_ref):
    lb = l_ref[...].astype(jnp.bfloat16)
    x = jnp.dot(lb, featb_ref[...], preferred_element_type=jnp.float32)
    e = feat_blk_ref[...]
    a = jnp.concatenate([e + x, x * e], axis=1).astype(jnp.bfloat16)
    out = jax.lax.dot_general(a, wc_ref[...], (((1,), (1,)), ((), ())),
                              preferred_element_type=jnp.float32)
    o_ref[...] = out + bias_ref[...]


@jax.jit
def kernel(lap_matrix, eye_matrix, features, W1, b1, W2, b2):
    del eye_matrix
    n, d_in = features.shape
    d_out = W1.shape[0]
    feat_b = features.astype(jnp.bfloat16)
    w_cat = jnp.concatenate([W1, W2], axis=1).astype(jnp.bfloat16)
    bias = (b1 + b2).reshape(1, d_out)

    grid = (n // _BM,)
    return pl.pallas_call(
        _body,
        grid=grid,
        in_specs=[
            pl.BlockSpec((_BM, n), lambda i: (i, 0)),
            pl.BlockSpec((n, d_in), lambda i: (0, 0)),
            pl.BlockSpec((_BM, d_in), lambda i: (i, 0)),
            pl.BlockSpec((d_out, 2 * d_in), lambda i: (0, 0)),
            pl.BlockSpec((1, d_out), lambda i: (0, 0)),
        ],
        out_specs=pl.BlockSpec((_BM, d_out), lambda i: (i, 0)),
        out_shape=jax.ShapeDtypeStruct((n, d_out), jnp.float32),
    )(lap_matrix, feat_b, features, w_cat, bias)


# speedup vs baseline: 1.1980x; 1.1980x over previous
import jax
import jax.numpy as jnp
from jax.experimental import pallas as pl

_BM = 256


def _body(l0_ref, l1_ref, feat_full_ref, feat_blk_ref, w1_ref, w2_ref,
          bias_ref, o_ref):
    kh = l0_ref.shape[1]
    x = jnp.dot(l0_ref[...], feat_full_ref[0:kh, :],
                preferred_element_type=jnp.float32)
    x += jnp.dot(l1_ref[...], feat_full_ref[kh:2 * kh, :],
                 preferred_element_type=jnp.float32)
    e = feat_blk_ref[...]
    dn = (((1,), (1,)), ((), ()))
    part1 = jax.lax.dot_general(e + x, w1_ref[...], dn,
                                preferred_element_type=jnp.float32)
    part2 = jax.lax.dot_general(x * e, w2_ref[...], dn,
                                preferred_element_type=jnp.float32)
    o_ref[...] = part1 + part2 + bias_ref[...]


@jax.jit
def kernel(lap_matrix, eye_matrix, features, W1, b1, W2, b2):
    del eye_matrix
    n, d_in = features.shape
    d_out = W1.shape[0]
    bias = (b1 + b2).reshape(1, d_out)
    nh = n // 2

    grid = (n // _BM,)
    return pl.pallas_call(
        _body,
        grid=grid,
        in_specs=[
            pl.BlockSpec((_BM, nh), lambda i: (i, 0)),
            pl.BlockSpec((_BM, nh), lambda i: (i, 1)),
            pl.BlockSpec((n, d_in), lambda i: (0, 0)),
            pl.BlockSpec((_BM, d_in), lambda i: (i, 0)),
            pl.BlockSpec((d_out, d_in), lambda i: (0, 0)),
            pl.BlockSpec((d_out, d_in), lambda i: (0, 0)),
            pl.BlockSpec((1, d_out), lambda i: (0, 0)),
        ],
        out_specs=pl.BlockSpec((_BM, d_out), lambda i: (i, 0)),
        out_shape=jax.ShapeDtypeStruct((n, d_out), jnp.float32),
    )(lap_matrix, lap_matrix, features, features, W1, W2, bias)


# speedup vs baseline: 1.3565x; 1.1323x over previous
import jax
import jax.numpy as jnp
from jax.experimental import pallas as pl

_BM = 512


def _body(l0_ref, l1_ref, feat_full_ref, feat_blk_ref, w1_ref, w2_ref,
          bias_ref, o_ref):
    kh = l0_ref.shape[1]
    x = jnp.dot(l0_ref[...], feat_full_ref[0:kh, :],
                preferred_element_type=jnp.float32)
    x += jnp.dot(l1_ref[...], feat_full_ref[kh:2 * kh, :],
                 preferred_element_type=jnp.float32)
    e = feat_blk_ref[...]
    dn = (((1,), (1,)), ((), ()))
    part1 = jax.lax.dot_general(e + x, w1_ref[...], dn,
                                preferred_element_type=jnp.float32)
    part2 = jax.lax.dot_general(x * e, w2_ref[...], dn,
                                preferred_element_type=jnp.float32)
    o_ref[...] = part1 + part2 + bias_ref[...]


@jax.jit
def kernel(lap_matrix, eye_matrix, features, W1, b1, W2, b2):
    del eye_matrix
    n, d_in = features.shape
    d_out = W1.shape[0]
    bias = (b1 + b2).reshape(1, d_out)
    nh = n // 2

    grid = (n // _BM,)
    return pl.pallas_call(
        _body,
        grid=grid,
        in_specs=[
            pl.BlockSpec((_BM, nh), lambda i: (i, 0)),
            pl.BlockSpec((_BM, nh), lambda i: (i, 1)),
            pl.BlockSpec((n, d_in), lambda i: (0, 0)),
            pl.BlockSpec((_BM, d_in), lambda i: (i, 0)),
            pl.BlockSpec((d_out, d_in), lambda i: (0, 0)),
            pl.BlockSpec((d_out, d_in), lambda i: (0, 0)),
            pl.BlockSpec((1, d_out), lambda i: (0, 0)),
        ],
        out_specs=pl.BlockSpec((_BM, d_out), lambda i: (i, 0)),
        out_shape=jax.ShapeDtypeStruct((n, d_out), jnp.float32),
    )(lap_matrix, lap_matrix, features, features, W1, W2, bias)
